# baseline (speedup 1.0000x reference)
import jax
import jax.numpy as jnp
from jax import lax
from jax.experimental import pallas as pl
from jax.experimental.pallas import tpu as pltpu
from jax.experimental.pallas import tpu_sc as plsc

_NUM_PTS = 128
_K_SAMPLE = 32
_K_SCORE = 4
_SIGMA = 0.01
_FEAT = 128
_BIG = 1e30
_IBIG = 1 << 30
_NW = 32
_PPW = 4


def _prep_body(sidx_ref, noisyT_ref, W1_ref, b1_ref, W2_ref, b2_ref,
               S1a_ref, S1b_ref, sb1_ref, S2_ref, sb2_ref,
               F_ref, E_ref, samp_ref, rsq_ref):
    n = noisyT_ref.shape[2]
    px = noisyT_ref[0, 0:1, :]
    py = noisyT_ref[0, 1:2, :]
    pz = noisyT_ref[0, 2:3, :]
    sidx = sidx_ref[...]
    col = jax.lax.broadcasted_iota(jnp.int32, (_NUM_PTS, n), 1)
    smask = col == sidx
    zero = jnp.float32(0.0)
    sx = jnp.sum(jnp.where(smask, px, zero), axis=1, keepdims=True)
    sy = jnp.sum(jnp.where(smask, py, zero), axis=1, keepdims=True)
    sz = jnp.sum(jnp.where(smask, pz, zero), axis=1, keepdims=True)
    samp_ref[0, :, :] = jnp.concatenate([sx, sy, sz], axis=1)

    h1 = jnp.maximum(sx * W1_ref[0:1, :] + sy * W1_ref[1:2, :]
                     + sz * W1_ref[2:3, :] + b1_ref[...], zero)
    feat = jnp.dot(h1, W2_ref[...], preferred_element_type=jnp.float32) + b2_ref[...]
    zS = jnp.dot(feat, S1b_ref[...], preferred_element_type=jnp.float32) + sb1_ref[...]

    d1 = (sx - px) ** 2 + (sy - py) ** 2 + (sz - pz) ** 2
    m = None
    for k in range(_K_SAMPLE):
        m = jnp.min(d1, axis=1, keepdims=True)
        eq = d1 == m
        fi = jnp.min(jnp.where(eq, col, _IBIG), axis=1, keepdims=True)
        sel = col == fi
        d1 = jnp.where(sel, _BIG, d1)
        fx = jnp.sum(jnp.where(sel, px, zero), axis=1, keepdims=True)
        fy = jnp.sum(jnp.where(sel, py, zero), axis=1, keepdims=True)
        fz = jnp.sum(jnp.where(sel, pz, zero), axis=1, keepdims=True)
        F_ref[0, k * _NUM_PTS:(k + 1) * _NUM_PTS, :] = jnp.concatenate(
            [fx, fy, fz], axis=1)
        xh = (fx - sx) * S1a_ref[0:1, :] + (fy - sy) * S1a_ref[1:2, :] \
            + (fz - sz) * S1a_ref[2:3, :]
        h = jnp.maximum(xh + zS, zero)
        ek = jnp.dot(h, S2_ref[...], preferred_element_type=jnp.float32) + sb2_ref[...]
        E_ref[0, k * _NUM_PTS:(k + 1) * _NUM_PTS, :] = ek
    rsq_ref[0, :, :] = m


def _lane_permute(x, perm):
    dnums = jax.lax.GatherDimensionNumbers(
        offset_dims=(), collapsed_slice_dims=(0,), start_index_map=(0,))
    return jax.lax.gather(x, perm[:, None], dnums, (1,),
                          mode=jax.lax.GatherScatterMode.PROMISE_IN_BOUNDS)


def _lane_max(x):
    lanes = jax.lax.broadcasted_iota(jnp.int32, (16,), 0)
    for k in (8, 4, 2, 1):
        x = jnp.maximum(x, _lane_permute(x, lanes ^ k))
    return x


def _lane_min(x):
    lanes = jax.lax.broadcasted_iota(jnp.int32, (16,), 0)
    for k in (8, 4, 2, 1):
        x = jnp.minimum(x, _lane_permute(x, lanes ^ k))
    return x


def _sc_knn2(Ft_hbm, Et_hbm, cleanT_hbm, St_hbm, Rt_hbm, out_hbm,
             F_v, E_v, S_v, R_v, clean_v, keys_v, idxs_v, acc_v):
    wid = lax.axis_index("s") * 2 + lax.axis_index("c")
    pltpu.sync_copy(Ft_hbm.at[wid], F_v)
    pltpu.sync_copy(Et_hbm.at[wid], E_v)
    pltpu.sync_copy(St_hbm.at[wid], S_v)
    pltpu.sync_copy(Rt_hbm.at[wid], R_v)
    acc_v[...] = jnp.zeros((16,), jnp.float32)
    m_pts = cleanT_hbm.shape[1] // 3
    n_chunks = m_pts // 16
    n_groups = Ft_hbm.shape[1] // (3 * _K_SAMPLE) * 2

    def group_body(g, carry):
        b = g // (2 * _PPW)
        rem = g - b * (2 * _PPW)
        p = rem // 2
        h = rem - p * 2

        @pl.when(rem == 0)
        def _():
            pltpu.sync_copy(cleanT_hbm.at[b], clean_v)

        zi = jnp.zeros((16,), jnp.int32)
        base_q = (b * 3 * _PPW + p) * _K_SAMPLE + h * 16
        qx = F_v[pl.ds(base_q, 16)]
        qy = F_v[pl.ds(base_q + _PPW * _K_SAMPLE, 16)]
        qz = F_v[pl.ds(base_q + 2 * _PPW * _K_SAMPLE, 16)]
        soff = b * 3 * _PPW + p
        gx = S_v[pl.ds(soff * 16, 16)]
        gy = S_v[pl.ds((soff + _PPW) * 16, 16)]
        gz = S_v[pl.ds((soff + 2 * _PPW) * 16, 16)]
        rho2 = R_v[pl.ds((b * _PPW + p) * 16, 16)]

        for lvl in range(_K_SCORE):
            keys_v[pl.ds(lvl * 16, 16)] = jnp.full((16,), _BIG, jnp.float32)
            idxs_v[pl.ds(lvl * 16, 16)] = jnp.zeros((16,), jnp.int32)

        def block_body(bi, c1):
            tmax = _lane_max(keys_v[pl.ds(3 * 16, 16)])
            thr = 2.0 * (rho2 + tmax)

            def chunk_body(ci, c2):
                base = ci * 16
                cx = clean_v[pl.ds(base, 16)]
                cy = clean_v[pl.ds(base + m_pts, 16)]
                cz = clean_v[pl.ds(base + 2 * m_pts, 16)]
                dcx = cx - gx
                dcy = cy - gy
                dcz = cz - gz
                dc = dcx * dcx + dcy * dcy + dcz * dcz

                @pl.when(_lane_min(dc - thr)[0] <= 0.0)
                def _():
                    def cand_body(t, c3):
                        i0 = zi + (base + t)
                        cxi = plsc.load_gather(clean_v, [i0])
                        cyi = plsc.load_gather(clean_v, [i0 + m_pts])
                        czi = plsc.load_gather(clean_v, [i0 + 2 * m_pts])
                        ddx = qx - cxi
                        ddy = qy - cyi
                        ddz = qz - czi
                        d16 = ddx * ddx + ddy * ddy + ddz * ddz

                        @pl.when(_lane_min(d16 - keys_v[pl.ds(3 * 16, 16)])[0] < 0.0)
                        def _():
                            ck = d16
                            cidx = i0
                            for lvl in range(_K_SCORE):
                                bk = keys_v[pl.ds(lvl * 16, 16)]
                                bi = idxs_v[pl.ds(lvl * 16, 16)]
                                pr = ck < bk
                                keys_v[pl.ds(lvl * 16, 16)] = jnp.where(pr, ck, bk)
                                idxs_v[pl.ds(lvl * 16, 16)] = jnp.where(pr, cidx, bi)
                                ck = jnp.where(pr, bk, ck)
                                cidx = jnp.where(pr, bi, cidx)
                        return c3

                    lax.fori_loop(0, 16, cand_body, 0)
                return c2

            lax.fori_loop(bi * 16, (bi + 1) * 16, chunk_body, 0)
            return c1

        lax.fori_loop(0, n_chunks // 16, block_body, 0)

        nnx = jnp.zeros((16,), jnp.float32)
        nny = jnp.zeros((16,), jnp.float32)
        nnz = jnp.zeros((16,), jnp.float32)
        for lvl in range(_K_SCORE):
            ii = idxs_v[pl.ds(lvl * 16, 16)]
            nnx = nnx + plsc.load_gather(clean_v, [ii])
            nny = nny + plsc.load_gather(clean_v, [ii + m_pts])
            nnz = nnz + plsc.load_gather(clean_v, [ii + 2 * m_pts])
        inv = jnp.float32(1.0 / _K_SCORE)
        ex = E_v[pl.ds(base_q, 16)]
        ey = E_v[pl.ds(base_q + _PPW * _K_SAMPLE, 16)]
        ez = E_v[pl.ds(base_q + 2 * _PPW * _K_SAMPLE, 16)]
        dx = ex - (nnx * inv - qx)
        dy = ey - (nny * inv - qy)
        dz = ez - (nnz * inv - qz)
        acc_v[...] = acc_v[...] + dx * dx + dy * dy + dz * dz
        return carry

    lax.fori_loop(0, n_groups, group_body, 0)
    pltpu.sync_copy(acc_v, out_hbm.at[wid])


def _morton_sorted(clean_pc):
    q = jnp.clip(jnp.floor((clean_pc + 3.0) * (8.0 / 6.0)), 0.0, 7.0)
    q = q.astype(jnp.int32)
    key = jnp.zeros(clean_pc.shape[:2], jnp.int32)
    for i in range(3):
        key = (key
               | (((q[..., 0] >> i) & 1) << (3 * i + 2))
               | (((q[..., 1] >> i) & 1) << (3 * i + 1))
               | (((q[..., 2] >> i) & 1) << (3 * i)))
    order = jnp.argsort(key, axis=1)
    return jnp.take_along_axis(clean_pc, order[..., None], axis=1)


def kernel(noisy_pc, clean_pc, W1, b1, W2, b2, S1, sb1, S2, sb2):
    B, N, _ = noisy_pc.shape
    M = clean_pc.shape[1]
    Q = _NUM_PTS * _K_SAMPLE

    sidx = jax.random.permutation(jax.random.key(1), N)[:_NUM_PTS]
    sidx = sidx.astype(jnp.int32).reshape(_NUM_PTS, 1)
    noisyT = jnp.transpose(noisy_pc, (0, 2, 1))

    fixed = lambda *shape: pl.BlockSpec(shape, lambda b: (0,) * len(shape))
    F, E, samp, rsq = pl.pallas_call(
        _prep_body,
        grid=(B,),
        in_specs=[
            fixed(_NUM_PTS, 1),
            pl.BlockSpec((1, 3, N), lambda b: (b, 0, 0)),
            fixed(3, _FEAT), fixed(1, _FEAT),
            fixed(_FEAT, _FEAT), fixed(1, _FEAT),
            fixed(3, _FEAT), fixed(_FEAT, _FEAT), fixed(1, _FEAT),
            fixed(_FEAT, 3), fixed(1, 3),
        ],
        out_specs=[
            pl.BlockSpec((1, Q, 3), lambda b: (b, 0, 0)),
            pl.BlockSpec((1, Q, 3), lambda b: (b, 0, 0)),
            pl.BlockSpec((1, _NUM_PTS, 3), lambda b: (b, 0, 0)),
            pl.BlockSpec((1, _NUM_PTS, 1), lambda b: (b, 0, 0)),
        ],
        out_shape=[
            jax.ShapeDtypeStruct((B, Q, 3), jnp.float32),
            jax.ShapeDtypeStruct((B, Q, 3), jnp.float32),
            jax.ShapeDtypeStruct((B, _NUM_PTS, 3), jnp.float32),
            jax.ShapeDtypeStruct((B, _NUM_PTS, 1), jnp.float32),
        ],
    )(sidx, noisyT, W1, b1.reshape(1, _FEAT), W2, b2.reshape(1, _FEAT),
      S1[:3], S1[3:], sb1.reshape(1, _FEAT), S2, sb2.reshape(1, 3))

    F4 = F.reshape(B, _K_SAMPLE, _NUM_PTS, 3)
    Ft = (F4.transpose(2, 0, 3, 1)
          .reshape(_NW, _PPW, B, 3, _K_SAMPLE)
          .transpose(0, 2, 3, 1, 4)
          .reshape(_NW, B * 3 * _PPW * _K_SAMPLE))
    E4 = E.reshape(B, _K_SAMPLE, _NUM_PTS, 3)
    Et = (E4.transpose(2, 0, 3, 1)
          .reshape(_NW, _PPW, B, 3, _K_SAMPLE)
          .transpose(0, 2, 3, 1, 4)
          .reshape(_NW, B * 3 * _PPW * _K_SAMPLE))
    St = (samp.transpose(1, 0, 2)
          .reshape(_NW, _PPW, B, 3)
          .transpose(0, 2, 3, 1)
          .reshape(_NW, B * 3 * _PPW))
    St = jnp.broadcast_to(St[:, :, None], (_NW, B * 3 * _PPW, 16)) \
        .reshape(_NW, B * 3 * _PPW * 16)
    Rt = (rsq.reshape(B, _NUM_PTS).transpose(1, 0)
          .reshape(_NW, _PPW, B)
          .transpose(0, 2, 1)
          .reshape(_NW, B * _PPW))
    Rt = jnp.broadcast_to(Rt[:, :, None], (_NW, B * _PPW, 16)) \
        .reshape(_NW, B * _PPW * 16)
    cleanT = jnp.transpose(_morton_sorted(clean_pc), (0, 2, 1)) \
        .reshape(B, 3 * M)

    mesh = plsc.VectorSubcoreMesh(core_axis_name="c", subcore_axis_name="s")
    parts = pl.kernel(
        _sc_knn2,
        out_type=jax.ShapeDtypeStruct((_NW, 16), jnp.float32),
        mesh=mesh,
        compiler_params=pltpu.CompilerParams(needs_layout_passes=False),
        scratch_types=[
            pltpu.VMEM((B * 3 * _PPW * _K_SAMPLE,), jnp.float32),
            pltpu.VMEM((B * 3 * _PPW * _K_SAMPLE,), jnp.float32),
            pltpu.VMEM((B * 3 * _PPW * 16,), jnp.float32),
            pltpu.VMEM((B * _PPW * 16,), jnp.float32),
            pltpu.VMEM((3 * M,), jnp.float32),
            pltpu.VMEM((_K_SCORE * 16,), jnp.float32),
            pltpu.VMEM((_K_SCORE * 16,), jnp.int32),
            pltpu.VMEM((16,), jnp.float32),
        ],
    )(Ft, Et, cleanT, St, Rt)

    denom = B * _NUM_PTS * _K_SAMPLE
    return 0.5 * (1.0 / _SIGMA) * jnp.sum(parts) / denom


# speedup vs baseline: 2.0586x; 2.0586x over previous
import jax
import jax.numpy as jnp
from jax import lax
from jax.experimental import pallas as pl
from jax.experimental.pallas import tpu as pltpu
from jax.experimental.pallas import tpu_sc as plsc

_NUM_PTS = 128
_K_SAMPLE = 32
_K_SCORE = 4
_SIGMA = 0.01
_FEAT = 128
_BIG = 1e30
_IBIG = 1 << 30
_NW = 32
_PPW = 4
_MF = 6


def _prep_body(sidx_ref, noisyT_ref, W1_ref, b1_ref, W2_ref, b2_ref,
               S1a_ref, S1b_ref, sb1_ref, S2_ref, sb2_ref,
               F_ref, E_ref, cen_ref, rad_ref):
    n = noisyT_ref.shape[2]
    px = noisyT_ref[0, 0:1, :]
    py = noisyT_ref[0, 1:2, :]
    pz = noisyT_ref[0, 2:3, :]
    sidx = sidx_ref[...]
    col = jax.lax.broadcasted_iota(jnp.int32, (_NUM_PTS, n), 1)
    smask = col == sidx
    zero = jnp.float32(0.0)
    sx = jnp.sum(jnp.where(smask, px, zero), axis=1, keepdims=True)
    sy = jnp.sum(jnp.where(smask, py, zero), axis=1, keepdims=True)
    sz = jnp.sum(jnp.where(smask, pz, zero), axis=1, keepdims=True)

    h1 = jnp.maximum(sx * W1_ref[0:1, :] + sy * W1_ref[1:2, :]
                     + sz * W1_ref[2:3, :] + b1_ref[...], zero)
    feat = jnp.dot(h1, W2_ref[...], preferred_element_type=jnp.float32) + b2_ref[...]
    zS = jnp.dot(feat, S1b_ref[...], preferred_element_type=jnp.float32) + sb1_ref[...]

    d1 = (sx - px) ** 2 + (sy - py) ** 2 + (sz - pz) ** 2
    csum = [[zero] * 3, [zero] * 3]
    for k in range(_K_SAMPLE):
        m = jnp.min(d1, axis=1, keepdims=True)
        eq = d1 == m
        fi = jnp.min(jnp.where(eq, col, _IBIG), axis=1, keepdims=True)
        sel = col == fi
        d1 = jnp.where(sel, _BIG, d1)
        fx = jnp.sum(jnp.where(sel, px, zero), axis=1, keepdims=True)
        fy = jnp.sum(jnp.where(sel, py, zero), axis=1, keepdims=True)
        fz = jnp.sum(jnp.where(sel, pz, zero), axis=1, keepdims=True)
        F_ref[0, k * _NUM_PTS:(k + 1) * _NUM_PTS, :] = jnp.concatenate(
            [fx, fy, fz], axis=1)
        hh = k // 16
        csum[hh] = [csum[hh][0] + fx, csum[hh][1] + fy, csum[hh][2] + fz]
        xh = (fx - sx) * S1a_ref[0:1, :] + (fy - sy) * S1a_ref[1:2, :] \
            + (fz - sz) * S1a_ref[2:3, :]
        h = jnp.maximum(xh + zS, zero)
        ek = jnp.dot(h, S2_ref[...], preferred_element_type=jnp.float32) + sb2_ref[...]
        E_ref[0, k * _NUM_PTS:(k + 1) * _NUM_PTS, :] = ek

    rads = []
    for hh in range(2):
        cx = csum[hh][0] * (1.0 / 16.0)
        cy = csum[hh][1] * (1.0 / 16.0)
        cz = csum[hh][2] * (1.0 / 16.0)
        cen_ref[0, :, 3 * hh:3 * hh + 3] = jnp.concatenate([cx, cy, cz], axis=1)
        r2 = jnp.zeros((_NUM_PTS, 1), jnp.float32)
        for k in range(hh * 16, (hh + 1) * 16):
            fx = F_ref[0, k * _NUM_PTS:(k + 1) * _NUM_PTS, 0:1]
            fy = F_ref[0, k * _NUM_PTS:(k + 1) * _NUM_PTS, 1:2]
            fz = F_ref[0, k * _NUM_PTS:(k + 1) * _NUM_PTS, 2:3]
            r2 = jnp.maximum(
                r2, (fx - cx) ** 2 + (fy - cy) ** 2 + (fz - cz) ** 2)
        r2 = r2 * 1.0002 + 1e-12
        rads.append(r2)
        rads.append(jnp.sqrt(r2) * 1.0001)
    rad_ref[0, :, :] = jnp.concatenate(rads, axis=1)


def _lane_permute(x, perm):
    dnums = jax.lax.GatherDimensionNumbers(
        offset_dims=(), collapsed_slice_dims=(0,), start_index_map=(0,))
    return jax.lax.gather(x, perm[:, None], dnums, (1,),
                          mode=jax.lax.GatherScatterMode.PROMISE_IN_BOUNDS)


def _lane_max(x):
    lanes = jax.lax.broadcasted_iota(jnp.int32, (16,), 0)
    for k in (8, 4, 2, 1):
        x = jnp.maximum(x, _lane_permute(x, lanes ^ k))
    return x


def _lane_min(x):
    lanes = jax.lax.broadcasted_iota(jnp.int32, (16,), 0)
    for k in (8, 4, 2, 1):
        x = jnp.minimum(x, _lane_permute(x, lanes ^ k))
    return x


def _sqrt_upper(x):
    u = jax.lax.bitcast_convert_type(x, jnp.int32)
    u = 0x5F3759DF - jax.lax.shift_right_logical(u, 1)
    y = jax.lax.bitcast_convert_type(u, jnp.float32)
    for _ in range(3):
        y = y * (1.5 - 0.5 * x * y * y)
    return x * y * 1.0002


def _sc_knn2(Ft_hbm, Et_hbm, cleanT_hbm, Mt_hbm, out_hbm,
             F_v, E_v, M_v, clean_v, keys_v, idxs_v, acc_v):
    wid = lax.axis_index("s") * 2 + lax.axis_index("c")
    pltpu.sync_copy(Ft_hbm.at[wid], F_v)
    pltpu.sync_copy(Et_hbm.at[wid], E_v)
    pltpu.sync_copy(Mt_hbm.at[wid], M_v)
    acc_v[...] = jnp.zeros((16,), jnp.float32)
    m_pts = cleanT_hbm.shape[1] // 3
    n_chunks = m_pts // 16
    n_groups = Ft_hbm.shape[1] // (3 * _K_SAMPLE) * 2

    def group_body(g, carry):
        b = g // (2 * _PPW)
        rem = g - b * (2 * _PPW)
        p = rem // 2
        h = rem - p * 2

        @pl.when(rem == 0)
        def _():
            pltpu.sync_copy(cleanT_hbm.at[b], clean_v)

        zi = jnp.zeros((16,), jnp.int32)
        base_q = (b * 3 * _PPW + p) * _K_SAMPLE + h * 16
        qx = F_v[pl.ds(base_q, 16)]
        qy = F_v[pl.ds(base_q + _PPW * _K_SAMPLE, 16)]
        qz = F_v[pl.ds(base_q + 2 * _PPW * _K_SAMPLE, 16)]
        moff = ((b * _PPW + p) * 2 + h) * (_MF * 16)
        gx = M_v[pl.ds(moff, 16)]
        gy = M_v[pl.ds(moff + 16, 16)]
        gz = M_v[pl.ds(moff + 32, 16)]
        rho2 = M_v[pl.ds(moff + 48, 16)]
        rho = M_v[pl.ds(moff + 64, 16)]
        start = M_v[pl.ds(moff + 80, 16)][0].astype(jnp.int32)

        for lvl in range(_K_SCORE):
            keys_v[pl.ds(lvl * 16, 16)] = jnp.full((16,), _BIG, jnp.float32)
            idxs_v[pl.ds(lvl * 16, 16)] = jnp.zeros((16,), jnp.int32)

        def chunk_body(cc, c2):
            ci = start + cc
            ci = ci - jnp.where(ci >= n_chunks, n_chunks, 0)
            base = ci * 16
            cx = clean_v[pl.ds(base, 16)]
            cy = clean_v[pl.ds(base + m_pts, 16)]
            cz = clean_v[pl.ds(base + 2 * m_pts, 16)]
            dcx = cx - gx
            dcy = cy - gy
            dcz = cz - gz
            dc = dcx * dcx + dcy * dcy + dcz * dcz
            tmax = _lane_max(keys_v[pl.ds(3 * 16, 16)])
            thr = rho2 + tmax + 2.0 * rho * _sqrt_upper(tmax)

            @pl.when(_lane_min(dc - thr)[0] <= 0.0)
            def _():
                def cand_body(t, c3):
                    i0 = zi + (base + t)
                    cxi = plsc.load_gather(clean_v, [i0])
                    cyi = plsc.load_gather(clean_v, [i0 + m_pts])
                    czi = plsc.load_gather(clean_v, [i0 + 2 * m_pts])
                    ddx = qx - cxi
                    ddy = qy - cyi
                    ddz = qz - czi
                    ck = ddx * ddx + ddy * ddy + ddz * ddz
                    cidx = i0
                    for lvl in range(_K_SCORE):
                        bk = keys_v[pl.ds(lvl * 16, 16)]
                        bi = idxs_v[pl.ds(lvl * 16, 16)]
                        pr = ck < bk
                        keys_v[pl.ds(lvl * 16, 16)] = jnp.where(pr, ck, bk)
                        idxs_v[pl.ds(lvl * 16, 16)] = jnp.where(pr, cidx, bi)
                        ck = jnp.where(pr, bk, ck)
                        cidx = jnp.where(pr, bi, cidx)
                    return c3

                lax.fori_loop(0, 16, cand_body, 0)
            return c2

        lax.fori_loop(0, n_chunks, chunk_body, 0)

        nnx = jnp.zeros((16,), jnp.float32)
        nny = jnp.zeros((16,), jnp.float32)
        nnz = jnp.zeros((16,), jnp.float32)
        for lvl in range(_K_SCORE):
            ii = idxs_v[pl.ds(lvl * 16, 16)]
            nnx = nnx + plsc.load_gather(clean_v, [ii])
            nny = nny + plsc.load_gather(clean_v, [ii + m_pts])
            nnz = nnz + plsc.load_gather(clean_v, [ii + 2 * m_pts])
        inv = jnp.float32(1.0 / _K_SCORE)
        ex = E_v[pl.ds(base_q, 16)]
        ey = E_v[pl.ds(base_q + _PPW * _K_SAMPLE, 16)]
        ez = E_v[pl.ds(base_q + 2 * _PPW * _K_SAMPLE, 16)]
        dx = ex - (nnx * inv - qx)
        dy = ey - (nny * inv - qy)
        dz = ez - (nnz * inv - qz)
        acc_v[...] = acc_v[...] + dx * dx + dy * dy + dz * dz
        return carry

    lax.fori_loop(0, n_groups, group_body, 0)
    pltpu.sync_copy(acc_v, out_hbm.at[wid])


def _bsp_order(clean_pc):
    B, M, _ = clean_pc.shape
    ox = jnp.argsort(clean_pc[..., 0], axis=1)
    s1 = jnp.take_along_axis(clean_pc, ox[..., None], 1)
    r1 = s1.reshape(B, 16, M // 16, 3)
    oy = jnp.argsort(r1[..., 1], axis=2)
    r2 = jnp.take_along_axis(r1, oy[..., None], 2)
    r3 = r2.reshape(B, 16, 16, M // 256, 3)
    oz = jnp.argsort(r3[..., 2], axis=3)
    r4 = jnp.take_along_axis(r3, oz[..., None], 3)
    xb = r1[:, :, 0, 0]
    yb = r2.reshape(B, 16, 16, M // 256, 3)[:, :, :, 0, 1]
    return r4.reshape(B, M, 3), xb, yb


def kernel(noisy_pc, clean_pc, W1, b1, W2, b2, S1, sb1, S2, sb2):
    B, N, _ = noisy_pc.shape
    M = clean_pc.shape[1]
    Q = _NUM_PTS * _K_SAMPLE

    sidx = jax.random.permutation(jax.random.key(1), N)[:_NUM_PTS]
    sidx = sidx.astype(jnp.int32).reshape(_NUM_PTS, 1)
    noisyT = jnp.transpose(noisy_pc, (0, 2, 1))

    fixed = lambda *shape: pl.BlockSpec(shape, lambda b: (0,) * len(shape))
    F, E, cen, rad = pl.pallas_call(
        _prep_body,
        grid=(B,),
        in_specs=[
            fixed(_NUM_PTS, 1),
            pl.BlockSpec((1, 3, N), lambda b: (b, 0, 0)),
            fixed(3, _FEAT), fixed(1, _FEAT),
            fixed(_FEAT, _FEAT), fixed(1, _FEAT),
            fixed(3, _FEAT), fixed(_FEAT, _FEAT), fixed(1, _FEAT),
            fixed(_FEAT, 3), fixed(1, 3),
        ],
        out_specs=[
            pl.BlockSpec((1, Q, 3), lambda b: (b, 0, 0)),
            pl.BlockSpec((1, Q, 3), lambda b: (b, 0, 0)),
            pl.BlockSpec((1, _NUM_PTS, 6), lambda b: (b, 0, 0)),
            pl.BlockSpec((1, _NUM_PTS, 4), lambda b: (b, 0, 0)),
        ],
        out_shape=[
            jax.ShapeDtypeStruct((B, Q, 3), jnp.float32),
            jax.ShapeDtypeStruct((B, Q, 3), jnp.float32),
            jax.ShapeDtypeStruct((B, _NUM_PTS, 6), jnp.float32),
            jax.ShapeDtypeStruct((B, _NUM_PTS, 4), jnp.float32),
        ],
    )(sidx, noisyT, W1, b1.reshape(1, _FEAT), W2, b2.reshape(1, _FEAT),
      S1[:3], S1[3:], sb1.reshape(1, _FEAT), S2, sb2.reshape(1, 3))

    clean_s, xb, yb = _bsp_order(clean_pc)
    cleanT = jnp.transpose(clean_s, (0, 2, 1)).reshape(B, 3 * M)

    cgx = cen[..., 0::3]
    cgy = cen[..., 1::3]
    ii = jnp.clip(jnp.sum(xb[:, None, None, :] <= cgx[..., None], -1) - 1,
                  0, 15)
    yb_g = yb[jnp.arange(B)[:, None, None], ii]
    jj = jnp.clip(jnp.sum(yb_g <= cgy[..., None], -1) - 1, 0, 15)
    startc = (ii * 16 + jj).astype(jnp.float32)

    def tileflat(a):
        return (a.transpose(2, 0, 3, 1)
                .reshape(_NW, _PPW, B, 3, _K_SAMPLE)
                .transpose(0, 2, 3, 1, 4)
                .reshape(_NW, B * 3 * _PPW * _K_SAMPLE))
    Ft = tileflat(F.reshape(B, _K_SAMPLE, _NUM_PTS, 3))
    Et = tileflat(E.reshape(B, _K_SAMPLE, _NUM_PTS, 3))

    meta = jnp.stack([cen[..., 0::3], cen[..., 1::3], cen[..., 2::3],
                      rad[..., 0::2], rad[..., 1::2], startc],
                     axis=-1)
    Mt = (meta.transpose(1, 0, 2, 3)
          .reshape(_NW, _PPW, B, 2, _MF)
          .transpose(0, 2, 1, 3, 4)
          .reshape(_NW, B * _PPW * 2 * _MF))
    Mt = jnp.broadcast_to(Mt[:, :, None], (_NW, B * _PPW * 2 * _MF, 16)) \
        .reshape(_NW, B * _PPW * 2 * _MF * 16)

    mesh = plsc.VectorSubcoreMesh(core_axis_name="c", subcore_axis_name="s")
    parts = pl.kernel(
        _sc_knn2,
        out_type=jax.ShapeDtypeStruct((_NW, 16), jnp.float32),
        mesh=mesh,
        compiler_params=pltpu.CompilerParams(needs_layout_passes=False),
        scratch_types=[
            pltpu.VMEM((B * 3 * _PPW * _K_SAMPLE,), jnp.float32),
            pltpu.VMEM((B * 3 * _PPW * _K_SAMPLE,), jnp.float32),
            pltpu.VMEM((B * _PPW * 2 * _MF * 16,), jnp.float32),
            pltpu.VMEM((3 * M,), jnp.float32),
            pltpu.VMEM((_K_SCORE * 16,), jnp.float32),
            pltpu.VMEM((_K_SCORE * 16,), jnp.int32),
            pltpu.VMEM((16,), jnp.float32),
        ],
    )(Ft, Et, cleanT, Mt)

    denom = B * _NUM_PTS * _K_SAMPLE
    return 0.5 * (1.0 / _SIGMA) * jnp.sum(parts) / denom


# speedup vs baseline: 2.1823x; 1.0601x over previous
import jax
import jax.numpy as jnp
from jax import lax
from jax.experimental import pallas as pl
from jax.experimental.pallas import tpu as pltpu
from jax.experimental.pallas import tpu_sc as plsc

_NUM_PTS = 128
_K_SAMPLE = 32
_K_SCORE = 4
_SIGMA = 0.01
_FEAT = 128
_BIG = 1e30
_IBIG = 1 << 30
_NW = 32
_PPW = 4
_MF = 6


def _prep_body(sidx_ref, noisyT_ref, W1_ref, b1_ref, W2_ref, b2_ref,
               S1a_ref, S1b_ref, sb1_ref, S2_ref, sb2_ref,
               F_ref, E_ref, cen_ref, rad_ref):
    n = noisyT_ref.shape[2]
    px = noisyT_ref[0, 0:1, :]
    py = noisyT_ref[0, 1:2, :]
    pz = noisyT_ref[0, 2:3, :]
    sidx = sidx_ref[...]
    col = jax.lax.broadcasted_iota(jnp.int32, (_NUM_PTS, n), 1)
    smask = col == sidx
    zero = jnp.float32(0.0)
    sx = jnp.sum(jnp.where(smask, px, zero), axis=1, keepdims=True)
    sy = jnp.sum(jnp.where(smask, py, zero), axis=1, keepdims=True)
    sz = jnp.sum(jnp.where(smask, pz, zero), axis=1, keepdims=True)

    h1 = jnp.maximum(sx * W1_ref[0:1, :] + sy * W1_ref[1:2, :]
                     + sz * W1_ref[2:3, :] + b1_ref[...], zero)
    feat = jnp.dot(h1, W2_ref[...], preferred_element_type=jnp.float32) + b2_ref[...]
    zS = jnp.dot(feat, S1b_ref[...], preferred_element_type=jnp.float32) + sb1_ref[...]

    d1 = (sx - px) ** 2 + (sy - py) ** 2 + (sz - pz) ** 2
    csum = [[zero] * 3, [zero] * 3]
    for k in range(_K_SAMPLE):
        m = jnp.min(d1, axis=1, keepdims=True)
        eq = d1 == m
        fi = jnp.min(jnp.where(eq, col, _IBIG), axis=1, keepdims=True)
        sel = col == fi
        d1 = jnp.where(sel, _BIG, d1)
        fx = jnp.sum(jnp.where(sel, px, zero), axis=1, keepdims=True)
        fy = jnp.sum(jnp.where(sel, py, zero), axis=1, keepdims=True)
        fz = jnp.sum(jnp.where(sel, pz, zero), axis=1, keepdims=True)
        F_ref[0, k * _NUM_PTS:(k + 1) * _NUM_PTS, :] = jnp.concatenate(
            [fx, fy, fz], axis=1)
        hh = k // 16
        csum[hh] = [csum[hh][0] + fx, csum[hh][1] + fy, csum[hh][2] + fz]
        xh = (fx - sx) * S1a_ref[0:1, :] + (fy - sy) * S1a_ref[1:2, :] \
            + (fz - sz) * S1a_ref[2:3, :]
        h = jnp.maximum(xh + zS, zero)
        ek = jnp.dot(h, S2_ref[...], preferred_element_type=jnp.float32) + sb2_ref[...]
        E_ref[0, k * _NUM_PTS:(k + 1) * _NUM_PTS, :] = ek

    rads = []
    for hh in range(2):
        cx = csum[hh][0] * (1.0 / 16.0)
        cy = csum[hh][1] * (1.0 / 16.0)
        cz = csum[hh][2] * (1.0 / 16.0)
        cen_ref[0, :, 3 * hh:3 * hh + 3] = jnp.concatenate([cx, cy, cz], axis=1)
        r2 = jnp.zeros((_NUM_PTS, 1), jnp.float32)
        for k in range(hh * 16, (hh + 1) * 16):
            fx = F_ref[0, k * _NUM_PTS:(k + 1) * _NUM_PTS, 0:1]
            fy = F_ref[0, k * _NUM_PTS:(k + 1) * _NUM_PTS, 1:2]
            fz = F_ref[0, k * _NUM_PTS:(k + 1) * _NUM_PTS, 2:3]
            r2 = jnp.maximum(
                r2, (fx - cx) ** 2 + (fy - cy) ** 2 + (fz - cz) ** 2)
        r2 = r2 * 1.0002 + 1e-12
        rads.append(r2)
        rads.append(jnp.sqrt(r2) * 1.0001)
    rad_ref[0, :, :] = jnp.concatenate(rads, axis=1)


def _lane_permute(x, perm):
    dnums = jax.lax.GatherDimensionNumbers(
        offset_dims=(), collapsed_slice_dims=(0,), start_index_map=(0,))
    return jax.lax.gather(x, perm[:, None], dnums, (1,),
                          mode=jax.lax.GatherScatterMode.PROMISE_IN_BOUNDS)


def _lane_max(x):
    lanes = jax.lax.broadcasted_iota(jnp.int32, (16,), 0)
    for k in (8, 4, 2, 1):
        x = jnp.maximum(x, _lane_permute(x, lanes ^ k))
    return x


def _lane_min(x):
    lanes = jax.lax.broadcasted_iota(jnp.int32, (16,), 0)
    for k in (8, 4, 2, 1):
        x = jnp.minimum(x, _lane_permute(x, lanes ^ k))
    return x


def _sqrt_upper(x):
    u = jax.lax.bitcast_convert_type(x, jnp.int32)
    u = 0x5F3759DF - jax.lax.shift_right_logical(u, 1)
    y = jax.lax.bitcast_convert_type(u, jnp.float32)
    for _ in range(3):
        y = y * (1.5 - 0.5 * x * y * y)
    return x * y * 1.0002


def _sc_knn2(Ft_hbm, Et_hbm, cleanT_hbm, Mt_hbm, out_hbm,
             F_v, E_v, M_v, clean_v, keys_v, idxs_v, acc_v):
    wid = lax.axis_index("s") * 2 + lax.axis_index("c")
    pltpu.sync_copy(Ft_hbm.at[wid], F_v)
    pltpu.sync_copy(Et_hbm.at[wid], E_v)
    pltpu.sync_copy(Mt_hbm.at[wid], M_v)
    acc_v[...] = jnp.zeros((16,), jnp.float32)
    m_pts = cleanT_hbm.shape[1] // 3
    n_chunks = m_pts // 16
    n_groups = Ft_hbm.shape[1] // (3 * _K_SAMPLE) * 2

    def group_body(g, carry):
        b = g // (2 * _PPW)
        rem = g - b * (2 * _PPW)
        p = rem // 2
        h = rem - p * 2

        @pl.when(rem == 0)
        def _():
            pltpu.sync_copy(cleanT_hbm.at[b], clean_v)

        zi = jnp.zeros((16,), jnp.int32)
        base_q = (b * 3 * _PPW + p) * _K_SAMPLE + h * 16
        qx = F_v[pl.ds(base_q, 16)]
        qy = F_v[pl.ds(base_q + _PPW * _K_SAMPLE, 16)]
        qz = F_v[pl.ds(base_q + 2 * _PPW * _K_SAMPLE, 16)]
        moff = ((b * _PPW + p) * 2 + h) * (_MF * 16)
        gx = M_v[pl.ds(moff, 16)]
        gy = M_v[pl.ds(moff + 16, 16)]
        gz = M_v[pl.ds(moff + 32, 16)]
        rho2 = M_v[pl.ds(moff + 48, 16)]
        rho = M_v[pl.ds(moff + 64, 16)]
        start = M_v[pl.ds(moff + 80, 16)][0].astype(jnp.int32)

        for lvl in range(_K_SCORE):
            keys_v[pl.ds(lvl * 16, 16)] = jnp.full((16,), _BIG, jnp.float32)
            idxs_v[pl.ds(lvl * 16, 16)] = jnp.zeros((16,), jnp.int32)

        def chunk_body(cc, thr):
            ci = start + cc
            ci = ci - jnp.where(ci >= n_chunks, n_chunks, 0)
            base = ci * 16
            cx = clean_v[pl.ds(base, 16)]
            cy = clean_v[pl.ds(base + m_pts, 16)]
            cz = clean_v[pl.ds(base + 2 * m_pts, 16)]
            dcx = cx - gx
            dcy = cy - gy
            dcz = cz - gz
            dc = dcx * dcx + dcy * dcy + dcz * dcz

            def slow_path(_):
                def cand_body(t, c3):
                    i0 = zi + (base + t)
                    cxi = plsc.load_gather(clean_v, [i0])
                    cyi = plsc.load_gather(clean_v, [i0 + m_pts])
                    czi = plsc.load_gather(clean_v, [i0 + 2 * m_pts])
                    ddx = qx - cxi
                    ddy = qy - cyi
                    ddz = qz - czi
                    ck = ddx * ddx + ddy * ddy + ddz * ddz
                    cidx = i0
                    for lvl in range(_K_SCORE):
                        bk = keys_v[pl.ds(lvl * 16, 16)]
                        bi = idxs_v[pl.ds(lvl * 16, 16)]
                        pr = ck < bk
                        keys_v[pl.ds(lvl * 16, 16)] = jnp.where(pr, ck, bk)
                        idxs_v[pl.ds(lvl * 16, 16)] = jnp.where(pr, cidx, bi)
                        ck = jnp.where(pr, bk, ck)
                        cidx = jnp.where(pr, bi, cidx)
                    return c3

                lax.fori_loop(0, 16, cand_body, 0)
                tmax = _lane_max(keys_v[pl.ds(3 * 16, 16)])
                return rho2 + tmax + 2.0 * rho * _sqrt_upper(tmax)

            return lax.cond(_lane_min(dc - thr)[0] <= 0.0,
                            slow_path, lambda _: thr, 0)

        lax.fori_loop(0, n_chunks, chunk_body,
                      jnp.full((16,), _BIG, jnp.float32))

        nnx = jnp.zeros((16,), jnp.float32)
        nny = jnp.zeros((16,), jnp.float32)
        nnz = jnp.zeros((16,), jnp.float32)
        for lvl in range(_K_SCORE):
            ii = idxs_v[pl.ds(lvl * 16, 16)]
            nnx = nnx + plsc.load_gather(clean_v, [ii])
            nny = nny + plsc.load_gather(clean_v, [ii + m_pts])
            nnz = nnz + plsc.load_gather(clean_v, [ii + 2 * m_pts])
        inv = jnp.float32(1.0 / _K_SCORE)
        ex = E_v[pl.ds(base_q, 16)]
        ey = E_v[pl.ds(base_q + _PPW * _K_SAMPLE, 16)]
        ez = E_v[pl.ds(base_q + 2 * _PPW * _K_SAMPLE, 16)]
        dx = ex - (nnx * inv - qx)
        dy = ey - (nny * inv - qy)
        dz = ez - (nnz * inv - qz)
        acc_v[...] = acc_v[...] + dx * dx + dy * dy + dz * dz
        return carry

    lax.fori_loop(0, n_groups, group_body, 0)
    pltpu.sync_copy(acc_v, out_hbm.at[wid])


def _bsp_order(clean_pc):
    B, M, _ = clean_pc.shape
    ox = jnp.argsort(clean_pc[..., 0], axis=1)
    s1 = jnp.take_along_axis(clean_pc, ox[..., None], 1)
    r1 = s1.reshape(B, 16, M // 16, 3)
    oy = jnp.argsort(r1[..., 1], axis=2)
    r2 = jnp.take_along_axis(r1, oy[..., None], 2)
    r3 = r2.reshape(B, 16, 16, M // 256, 3)
    oz = jnp.argsort(r3[..., 2], axis=3)
    r4 = jnp.take_along_axis(r3, oz[..., None], 3)
    xb = r1[:, :, 0, 0]
    yb = r2.reshape(B, 16, 16, M // 256, 3)[:, :, :, 0, 1]
    return r4.reshape(B, M, 3), xb, yb


def kernel(noisy_pc, clean_pc, W1, b1, W2, b2, S1, sb1, S2, sb2):
    B, N, _ = noisy_pc.shape
    M = clean_pc.shape[1]
    Q = _NUM_PTS * _K_SAMPLE

    sidx = jax.random.permutation(jax.random.key(1), N)[:_NUM_PTS]
    sidx = sidx.astype(jnp.int32).reshape(_NUM_PTS, 1)
    noisyT = jnp.transpose(noisy_pc, (0, 2, 1))

    fixed = lambda *shape: pl.BlockSpec(shape, lambda b: (0,) * len(shape))
    F, E, cen, rad = pl.pallas_call(
        _prep_body,
        grid=(B,),
        in_specs=[
            fixed(_NUM_PTS, 1),
            pl.BlockSpec((1, 3, N), lambda b: (b, 0, 0)),
            fixed(3, _FEAT), fixed(1, _FEAT),
            fixed(_FEAT, _FEAT), fixed(1, _FEAT),
            fixed(3, _FEAT), fixed(_FEAT, _FEAT), fixed(1, _FEAT),
            fixed(_FEAT, 3), fixed(1, 3),
        ],
        out_specs=[
            pl.BlockSpec((1, Q, 3), lambda b: (b, 0, 0)),
            pl.BlockSpec((1, Q, 3), lambda b: (b, 0, 0)),
            pl.BlockSpec((1, _NUM_PTS, 6), lambda b: (b, 0, 0)),
            pl.BlockSpec((1, _NUM_PTS, 4), lambda b: (b, 0, 0)),
        ],
        out_shape=[
            jax.ShapeDtypeStruct((B, Q, 3), jnp.float32),
            jax.ShapeDtypeStruct((B, Q, 3), jnp.float32),
            jax.ShapeDtypeStruct((B, _NUM_PTS, 6), jnp.float32),
            jax.ShapeDtypeStruct((B, _NUM_PTS, 4), jnp.float32),
        ],
    )(sidx, noisyT, W1, b1.reshape(1, _FEAT), W2, b2.reshape(1, _FEAT),
      S1[:3], S1[3:], sb1.reshape(1, _FEAT), S2, sb2.reshape(1, 3))

    clean_s, xb, yb = _bsp_order(clean_pc)
    cleanT = jnp.transpose(clean_s, (0, 2, 1)).reshape(B, 3 * M)

    cgx = cen[..., 0::3]
    cgy = cen[..., 1::3]
    ii = jnp.clip(jnp.sum(xb[:, None, None, :] <= cgx[..., None], -1) - 1,
                  0, 15)
    yb_g = yb[jnp.arange(B)[:, None, None], ii]
    jj = jnp.clip(jnp.sum(yb_g <= cgy[..., None], -1) - 1, 0, 15)
    startc = (ii * 16 + jj).astype(jnp.float32)

    def tileflat(a):
        return (a.transpose(2, 0, 3, 1)
                .reshape(_NW, _PPW, B, 3, _K_SAMPLE)
                .transpose(0, 2, 3, 1, 4)
                .reshape(_NW, B * 3 * _PPW * _K_SAMPLE))
    Ft = tileflat(F.reshape(B, _K_SAMPLE, _NUM_PTS, 3))
    Et = tileflat(E.reshape(B, _K_SAMPLE, _NUM_PTS, 3))

    meta = jnp.stack([cen[..., 0::3], cen[..., 1::3], cen[..., 2::3],
                      rad[..., 0::2], rad[..., 1::2], startc],
                     axis=-1)
    Mt = (meta.transpose(1, 0, 2, 3)
          .reshape(_NW, _PPW, B, 2, _MF)
          .transpose(0, 2, 1, 3, 4)
          .reshape(_NW, B * _PPW * 2 * _MF))
    Mt = jnp.broadcast_to(Mt[:, :, None], (_NW, B * _PPW * 2 * _MF, 16)) \
        .reshape(_NW, B * _PPW * 2 * _MF * 16)

    mesh = plsc.VectorSubcoreMesh(core_axis_name="c", subcore_axis_name="s")
    parts = pl.kernel(
        _sc_knn2,
        out_type=jax.ShapeDtypeStruct((_NW, 16), jnp.float32),
        mesh=mesh,
        compiler_params=pltpu.CompilerParams(needs_layout_passes=False),
        scratch_types=[
            pltpu.VMEM((B * 3 * _PPW * _K_SAMPLE,), jnp.float32),
            pltpu.VMEM((B * 3 * _PPW * _K_SAMPLE,), jnp.float32),
            pltpu.VMEM((B * _PPW * 2 * _MF * 16,), jnp.float32),
            pltpu.VMEM((3 * M,), jnp.float32),
            pltpu.VMEM((_K_SCORE * 16,), jnp.float32),
            pltpu.VMEM((_K_SCORE * 16,), jnp.int32),
            pltpu.VMEM((16,), jnp.float32),
        ],
    )(Ft, Et, cleanT, Mt)

    denom = B * _NUM_PTS * _K_SAMPLE
    return 0.5 * (1.0 / _SIGMA) * jnp.sum(parts) / denom
